# baseline (speedup 1.0000x reference)
import jax
import jax.numpy as jnp
from jax import lax
from jax.experimental import pallas as pl
from jax.experimental.pallas import tpu as pltpu
from jax.experimental.pallas import tpu_sc as plsc

N = 10000
D = 128
E = 320000

NC = 2
NS = 16
NW = NC * NS

CH = 128
EP = 327680
EPW = EP // NW
NCH = EPW // CH
DPAD = 10200

NACC = 10240
RPT = NACC // NS
ZB = 128


def _make_agg(with_cnt: bool):
  out_type = [jax.ShapeDtypeStruct((NC, NACC, D), jnp.float32)]
  scratch = (
      [pltpu.VMEM((CH,), jnp.int32)] * 3 +
      [pltpu.VMEM((CH,), jnp.int32)] * 3 +
      [pltpu.VMEM((CH, D), jnp.float32)] * 2 +
      [pltpu.VMEM_SHARED((NACC, D), jnp.float32)] +
      [pltpu.SemaphoreType.DMA] * 5
  )
  if with_cnt:
    out_type.append(jax.ShapeDtypeStruct((NW, NACC), jnp.float32))
    scratch += [
        pltpu.VMEM((NACC,), jnp.float32),
    ]

  mesh = plsc.VectorSubcoreMesh(
      core_axis_name="c", subcore_axis_name="s",
      num_cores=NC, num_subcores=NS)

  def body(table_hbm, src_hbm, dst_hbm, z_hbm, zc_hbm, *refs):
    if with_cnt:
      (acc_out, cnt_out, s0, s1, s2, d0, d1, d2, r0_, r1_, accum,
       g0, g1, i0, i1, i2, hist) = refs
    else:
      (acc_out, s0, s1, s2, d0, d1, d2, r0_, r1_, accum,
       g0, g1, i0, i1, i2) = refs
    srcs, dsts, rows = [s0, s1, s2], [d0, d1, d2], [r0_, r1_]
    gsems, isems = [g0, g1], [i0, i1, i2]
    c = lax.axis_index("c")
    s = lax.axis_index("s")
    w = s * NC + c

    def issue_idx(k, ch):
      pltpu.async_copy(src_hbm.at[w].at[ch].at[0], srcs[k], isems[k])
      pltpu.async_copy(dst_hbm.at[w].at[ch].at[0], dsts[k], isems[k])

    def wait_idx(k, ch):
      pltpu.make_async_copy(src_hbm.at[w].at[ch].at[0], srcs[k],
                            isems[k]).wait()
      pltpu.make_async_copy(dst_hbm.at[w].at[ch].at[0], dsts[k],
                            isems[k]).wait()

    def issue_gather(r, k):
      pltpu.async_copy(table_hbm.at[srcs[k]], rows[r], gsems[r])

    def wait_gather(r, k):
      pltpu.make_async_copy(table_hbm.at[srcs[k]], rows[r],
                            gsems[r]).wait()

    def scatter(r, k):
      pltpu.sync_copy(rows[r], accum.at[dsts[k]], add=True)
      if with_cnt:
        ones16 = jnp.full((16,), 1.0, jnp.float32)
        for kk in range(CH // 16):
          idxv = dsts[k][pl.ds(kk * 16, 16)]
          plsc.addupdate_scatter(hist, [idxv], ones16)

    pltpu.sync_copy(z_hbm, rows[0])
    for b in range(RPT // ZB):
      pltpu.sync_copy(rows[0], accum.at[pl.ds(s * RPT + b * ZB, ZB)])
    if with_cnt:
      pltpu.sync_copy(zc_hbm, hist)
    plsc.subcore_barrier()

    issue_idx(0, 0)
    wait_idx(0, 0)
    issue_gather(0, 0)
    issue_idx(1, 1)

    def step(i, carry):
      base = 6 * i
      for k in range(6):
        ch = base + k
        r, rn = k % 2, (k + 1) % 2
        ik, i_n, i_p = k % 3, (k + 1) % 3, (k + 2) % 3
        wait_idx(i_n, ch + 1)
        issue_gather(rn, i_n)
        issue_idx(i_p, ch + 2)
        wait_gather(r, ik)
        scatter(r, ik)
      return carry
    lax.fori_loop(0, (NCH - 2) // 6, step, 0)

    wait_idx(1, NCH - 1)
    issue_gather(1, 1)
    wait_gather(0, 0)
    scatter(0, 0)
    wait_gather(1, 1)
    scatter(1, 1)
    plsc.subcore_barrier()

    for b in range(RPT // ZB):
      rr = s * RPT + b * ZB
      pltpu.sync_copy(accum.at[pl.ds(rr, ZB)], rows[b % 2])
      pltpu.sync_copy(rows[b % 2], acc_out.at[c].at[pl.ds(rr, ZB)])
    if with_cnt:
      pltpu.sync_copy(hist, cnt_out.at[w])

  return pl.kernel(
      body, out_type=out_type, mesh=mesh, scratch_types=scratch,
      compiler_params=pltpu.CompilerParams(needs_layout_passes=False))


_agg_with_cnt = _make_agg(True)
_agg_no_cnt = _make_agg(False)


def _mm(acc, cnt2, xin, wl, wr, b, relu):
  BN = 1024
  grid = (N + BN - 1) // BN

  def mmbody(acc_ref, cnt_ref, x_ref, wl_ref, wr_ref, b_ref, o_ref):
    cnt = jnp.sum(cnt_ref[...], axis=0)[:, None]
    ssum = acc_ref[0] + acc_ref[1]
    mean = ssum / jnp.maximum(cnt, 1.0)
    r = (jnp.dot(mean, wl_ref[...], preferred_element_type=jnp.float32)
         + b_ref[...]
         + jnp.dot(x_ref[...], wr_ref[...],
                   preferred_element_type=jnp.float32))
    o_ref[...] = jnp.maximum(r, 0.0) if relu else r

  return pl.pallas_call(
      mmbody,
      grid=(grid,),
      in_specs=[
          pl.BlockSpec((2, BN, D), lambda i: (0, i, 0)),
          pl.BlockSpec((NW, BN), lambda i: (0, i)),
          pl.BlockSpec((BN, D), lambda i: (i, 0)),
          pl.BlockSpec((D, D), lambda i: (0, 0)),
          pl.BlockSpec((D, D), lambda i: (0, 0)),
          pl.BlockSpec((1, D), lambda i: (0, 0)),
      ],
      out_specs=pl.BlockSpec((BN, D), lambda i: (i, 0)),
      out_shape=jax.ShapeDtypeStruct((N, D), jnp.float32),
  )(acc, cnt2, xin, wl, wr, b.reshape(1, D))


@jax.jit
def kernel(x, edge_index, W1_l, W1_r, b1, W2_l, W2_r, b2):
  pad = EP - E
  src = jnp.concatenate(
      [edge_index[0], jnp.zeros((pad,), jnp.int32)]).reshape(NW, NCH, 1, CH)
  dst = jnp.concatenate(
      [edge_index[1], jnp.full((pad,), DPAD, jnp.int32)]).reshape(
          NW, NCH, 1, CH)
  z_hbm = jnp.zeros((ZB, D), jnp.float32)
  zc_hbm = jnp.zeros((NACC,), jnp.float32)

  acc1, cnt2 = _agg_with_cnt(x, src, dst, z_hbm, zc_hbm)
  h = _mm(acc1, cnt2, x, W1_l, W1_r, b1, relu=True)
  (acc2,) = _agg_no_cnt(h, src, dst, z_hbm, zc_hbm)
  out = _mm(acc2, cnt2, h, W2_l, W2_r, b2, relu=False)
  return out


# speedup vs baseline: 3.4898x; 3.4898x over previous
import jax
import jax.numpy as jnp
from jax import lax
from jax.experimental import pallas as pl
from jax.experimental.pallas import tpu as pltpu
from jax.experimental.pallas import tpu_sc as plsc

N = 10000
D = 128
E = 320000

NC = 2
NS = 16
NW = NC * NS

EPW = E // NW
CH = 80
NCH = EPW // CH

NACC = 10240
RPT = NACC // NS
ZB = 80


def _make_agg(with_cnt: bool):
  out_type = [jax.ShapeDtypeStruct((NC, NACC, D), jnp.float32)]
  scratch = (
      [pltpu.VMEM((CH,), jnp.int32)] * 3 +
      [pltpu.VMEM((CH,), jnp.int32)] * 3 +
      [pltpu.VMEM((CH, D), jnp.float32)] * 3 +
      [pltpu.VMEM_SHARED((NACC, D), jnp.float32)] +
      [pltpu.SemaphoreType.DMA] * 6
  )
  if with_cnt:
    out_type.append(jax.ShapeDtypeStruct((NW, NACC), jnp.float32))
    scratch += [
        pltpu.VMEM((NACC,), jnp.float32),
    ]

  mesh = plsc.VectorSubcoreMesh(
      core_axis_name="c", subcore_axis_name="s",
      num_cores=NC, num_subcores=NS)

  def body(table_hbm, src_hbm, dst_hbm, z_hbm, zc_hbm, *refs):
    if with_cnt:
      (acc_out, cnt_out, s0, s1, s2, d0, d1, d2, r0_, r1_, r2_, accum,
       g0, g1, g2, i0, i1, i2, hist) = refs
    else:
      (acc_out, s0, s1, s2, d0, d1, d2, r0_, r1_, r2_, accum,
       g0, g1, g2, i0, i1, i2) = refs
    srcs, dsts, rows = [s0, s1, s2], [d0, d1, d2], [r0_, r1_, r2_]
    gsems, isems = [g0, g1, g2], [i0, i1, i2]
    c = lax.axis_index("c")
    s = lax.axis_index("s")
    w = s * NC + c

    def issue_idx(k, ch):
      pltpu.async_copy(src_hbm.at[w].at[ch].at[0], srcs[k], isems[k])
      pltpu.async_copy(dst_hbm.at[w].at[ch].at[0], dsts[k], isems[k])

    def wait_idx(k, ch):
      pltpu.make_async_copy(src_hbm.at[w].at[ch].at[0], srcs[k],
                            isems[k]).wait()
      pltpu.make_async_copy(dst_hbm.at[w].at[ch].at[0], dsts[k],
                            isems[k]).wait()

    def issue_gather(k):
      pltpu.async_copy(table_hbm.at[srcs[k]], rows[k], gsems[k])

    def wait_gather(k):
      pltpu.make_async_copy(table_hbm.at[srcs[k]], rows[k],
                            gsems[k]).wait()

    def scatter(k):
      pltpu.sync_copy(rows[k], accum.at[dsts[k]], add=True)
      if with_cnt:
        ones16 = jnp.full((16,), 1.0, jnp.float32)
        for kk in range(CH // 16):
          idxv = dsts[k][pl.ds(kk * 16, 16)]
          plsc.addupdate_scatter(hist, [idxv], ones16)

    if with_cnt:
      pltpu.async_copy(zc_hbm, hist, isems[0])
    pltpu.sync_copy(z_hbm, rows[0])
    for b in range(RPT // ZB):
      pltpu.async_copy(rows[0], accum.at[pl.ds(s * RPT + b * ZB, ZB)],
                       gsems[0])
    for b in range(RPT // ZB):
      pltpu.make_async_copy(rows[0], accum.at[pl.ds(s * RPT + b * ZB, ZB)],
                            gsems[0]).wait()
    if with_cnt:
      pltpu.make_async_copy(zc_hbm, hist, isems[0]).wait()
    plsc.subcore_barrier()

    issue_idx(0, 0)
    wait_idx(0, 0)
    issue_gather(0)
    issue_idx(1, 1)

    def step(i, carry):
      base = 3 * i
      for k in range(3):
        ch = base + k
        kn, kp = (k + 1) % 3, (k + 2) % 3
        wait_idx(kn, ch + 1)
        issue_gather(kn)
        issue_idx(kp, ch + 2)
        wait_gather(k)
        scatter(k)
      return carry
    lax.fori_loop(0, NCH // 3, step, 0)

    wait_idx(1, NCH - 1)
    issue_gather(1)
    wait_gather(0)
    scatter(0)
    wait_gather(1)
    scatter(1)
    plsc.subcore_barrier()

    if with_cnt:
      pltpu.async_copy(hist, cnt_out.at[w], isems[0])

    def _wr(sl, b):
      return pltpu.make_async_copy(
          rows[sl], acc_out.at[c].at[pl.ds(s * RPT + b * ZB, ZB)],
          gsems[sl])

    nb = RPT // ZB
    for b in range(nb):
      sl = b % 3
      if b >= 3:
        _wr(sl, b - 3).wait()
      rr = s * RPT + b * ZB
      pltpu.sync_copy(accum.at[pl.ds(rr, ZB)], rows[sl])
      pltpu.async_copy(rows[sl], acc_out.at[c].at[pl.ds(rr, ZB)],
                       gsems[sl])
    for b in range(nb - 3, nb):
      _wr(b % 3, b).wait()
    if with_cnt:
      pltpu.make_async_copy(hist, cnt_out.at[w], isems[0]).wait()

  return pl.kernel(
      body, out_type=out_type, mesh=mesh, scratch_types=scratch,
      compiler_params=pltpu.CompilerParams(needs_layout_passes=False))


_agg_with_cnt = _make_agg(True)
_agg_no_cnt = _make_agg(False)


def _mm(acc, cnt2, xin, wl, wr, b, relu):
  BN = 1024
  grid = (N + BN - 1) // BN

  def mmbody(acc_ref, cnt_ref, x_ref, wl_ref, wr_ref, b_ref, o_ref):
    cnt = jnp.sum(cnt_ref[...], axis=0)[:, None]
    ssum = acc_ref[0] + acc_ref[1]
    mean = ssum / jnp.maximum(cnt, 1.0)
    r = (jnp.dot(mean, wl_ref[...], preferred_element_type=jnp.float32)
         + b_ref[...]
         + jnp.dot(x_ref[...], wr_ref[...],
                   preferred_element_type=jnp.float32))
    o_ref[...] = jnp.maximum(r, 0.0) if relu else r

  return pl.pallas_call(
      mmbody,
      grid=(grid,),
      in_specs=[
          pl.BlockSpec((2, BN, D), lambda i: (0, i, 0)),
          pl.BlockSpec((NW, BN), lambda i: (0, i)),
          pl.BlockSpec((BN, D), lambda i: (i, 0)),
          pl.BlockSpec((D, D), lambda i: (0, 0)),
          pl.BlockSpec((D, D), lambda i: (0, 0)),
          pl.BlockSpec((1, D), lambda i: (0, 0)),
      ],
      out_specs=pl.BlockSpec((BN, D), lambda i: (i, 0)),
      out_shape=jax.ShapeDtypeStruct((N, D), jnp.float32),
  )(acc, cnt2, xin, wl, wr, b.reshape(1, D))


@jax.jit
def kernel(x, edge_index, W1_l, W1_r, b1, W2_l, W2_r, b2):
  src = edge_index[0].reshape(NW, NCH, 1, CH)
  dst = edge_index[1].reshape(NW, NCH, 1, CH)
  z_hbm = jnp.zeros((ZB, D), jnp.float32)
  zc_hbm = jnp.zeros((NACC,), jnp.float32)

  acc1, cnt2 = _agg_with_cnt(x, src, dst, z_hbm, zc_hbm)
  h = _mm(acc1, cnt2, x, W1_l, W1_r, b1, relu=True)
  (acc2,) = _agg_no_cnt(h, src, dst, z_hbm, zc_hbm)
  out = _mm(acc2, cnt2, h, W2_l, W2_r, b2, relu=False)
  return out


# speedup vs baseline: 3.6575x; 1.0481x over previous
import jax
import jax.numpy as jnp
from jax import lax
from jax.experimental import pallas as pl
from jax.experimental.pallas import tpu as pltpu
from jax.experimental.pallas import tpu_sc as plsc

N = 10000
D = 128
E = 320000

NC = 2
NS = 16
NW = NC * NS

EPW = E // NW
CH = 80
NCH = EPW // CH

NACC = 10240
RPT = NACC // NS
ZB = 80


def _make_agg(with_cnt: bool):
  out_type = [jax.ShapeDtypeStruct((NC, NACC, D), jnp.float32)]
  scratch = (
      [pltpu.VMEM((CH,), jnp.int32)] * 3 +
      [pltpu.VMEM((CH,), jnp.int32)] * 3 +
      [pltpu.VMEM((CH, D), jnp.float32)] * 3 +
      [pltpu.VMEM_SHARED((NACC, D), jnp.float32)] +
      [pltpu.SemaphoreType.DMA] * 9
  )
  if with_cnt:
    out_type.append(jax.ShapeDtypeStruct((NW, NACC), jnp.float32))
    scratch += [
        pltpu.VMEM((NACC,), jnp.float32),
    ]

  mesh = plsc.VectorSubcoreMesh(
      core_axis_name="c", subcore_axis_name="s",
      num_cores=NC, num_subcores=NS)

  def body(table_hbm, src_hbm, dst_hbm, z_hbm, zc_hbm, *refs):
    if with_cnt:
      (acc_out, cnt_out, s0, s1, s2, d0, d1, d2, r0_, r1_, r2_, accum,
       g0, g1, g2, i0, i1, i2, x0, x1, x2, hist) = refs
    else:
      (acc_out, s0, s1, s2, d0, d1, d2, r0_, r1_, r2_, accum,
       g0, g1, g2, i0, i1, i2, x0, x1, x2) = refs
    srcs, dsts, rows = [s0, s1, s2], [d0, d1, d2], [r0_, r1_, r2_]
    gsems, isems, ssems = [g0, g1, g2], [i0, i1, i2], [x0, x1, x2]
    c = lax.axis_index("c")
    s = lax.axis_index("s")
    w = s * NC + c

    def issue_idx(k, ch):
      pltpu.async_copy(src_hbm.at[w].at[ch].at[0], srcs[k], isems[k])
      pltpu.async_copy(dst_hbm.at[w].at[ch].at[0], dsts[k], isems[k])

    def wait_idx(k, ch):
      pltpu.make_async_copy(src_hbm.at[w].at[ch].at[0], srcs[k],
                            isems[k]).wait()
      pltpu.make_async_copy(dst_hbm.at[w].at[ch].at[0], dsts[k],
                            isems[k]).wait()

    def issue_gather(k):
      pltpu.async_copy(table_hbm.at[srcs[k]], rows[k], gsems[k])

    def wait_gather(k):
      pltpu.make_async_copy(table_hbm.at[srcs[k]], rows[k],
                            gsems[k]).wait()

    def scatter(k):
      pltpu.async_copy(rows[k], accum.at[dsts[k]], ssems[k], add=True)
      if with_cnt:
        ones16 = jnp.full((16,), 1.0, jnp.float32)
        for kk in range(CH // 16):
          idxv = dsts[k][pl.ds(kk * 16, 16)]
          plsc.addupdate_scatter(hist, [idxv], ones16)

    def wait_scatter(k):
      pltpu.make_async_copy(rows[k], accum.at[dsts[k]], ssems[k]).wait()

    if with_cnt:
      pltpu.async_copy(zc_hbm, hist, isems[0])
    pltpu.sync_copy(z_hbm, rows[0])
    for b in range(RPT // ZB):
      pltpu.async_copy(rows[0], accum.at[pl.ds(s * RPT + b * ZB, ZB)],
                       gsems[0])
    for b in range(RPT // ZB):
      pltpu.make_async_copy(rows[0], accum.at[pl.ds(s * RPT + b * ZB, ZB)],
                            gsems[0]).wait()
    if with_cnt:
      pltpu.make_async_copy(zc_hbm, hist, isems[0]).wait()
    plsc.subcore_barrier()

    issue_idx(0, 0)
    wait_idx(0, 0)
    issue_gather(0)
    issue_idx(1, 1)
    wait_idx(1, 1)
    issue_gather(1)
    issue_idx(2, 2)
    wait_gather(0)
    scatter(0)

    def step(i, carry):
      base = 3 * i + 1
      for k in range(3):
        ch = base + k
        kc = (1 + k) % 3
        kn = (2 + k) % 3
        kp = k % 3
        wait_idx(kn, ch + 1)
        issue_gather(kn)
        wait_scatter(kp)
        issue_idx(kp, jnp.minimum(ch + 2, NCH - 1))
        wait_gather(kc)
        scatter(kc)
      return carry
    lax.fori_loop(0, (NCH - 2) // 3, step, 0)

    wait_scatter(0)
    wait_gather(1)
    scatter(1)
    wait_scatter(1)
    wait_idx(2, NCH - 1)
    plsc.subcore_barrier()

    if with_cnt:
      pltpu.async_copy(hist, cnt_out.at[w], isems[0])

    def _wr(sl, b):
      return pltpu.make_async_copy(
          rows[sl], acc_out.at[c].at[pl.ds(s * RPT + b * ZB, ZB)],
          gsems[sl])

    nb = RPT // ZB
    for b in range(nb):
      sl = b % 3
      if b >= 3:
        _wr(sl, b - 3).wait()
      rr = s * RPT + b * ZB
      pltpu.sync_copy(accum.at[pl.ds(rr, ZB)], rows[sl])
      pltpu.async_copy(rows[sl], acc_out.at[c].at[pl.ds(rr, ZB)],
                       gsems[sl])
    for b in range(nb - 3, nb):
      _wr(b % 3, b).wait()
    if with_cnt:
      pltpu.make_async_copy(hist, cnt_out.at[w], isems[0]).wait()

  return pl.kernel(
      body, out_type=out_type, mesh=mesh, scratch_types=scratch,
      compiler_params=pltpu.CompilerParams(needs_layout_passes=False))


_agg_with_cnt = _make_agg(True)
_agg_no_cnt = _make_agg(False)


def _mm(acc, cnt2, xin, wl, wr, b, relu):
  BN = 1024
  grid = (N + BN - 1) // BN

  def mmbody(acc_ref, cnt_ref, x_ref, wl_ref, wr_ref, b_ref, o_ref):
    cnt = jnp.sum(cnt_ref[...], axis=0)[:, None]
    ssum = acc_ref[0] + acc_ref[1]
    mean = ssum / jnp.maximum(cnt, 1.0)
    r = (jnp.dot(mean, wl_ref[...], preferred_element_type=jnp.float32)
         + b_ref[...]
         + jnp.dot(x_ref[...], wr_ref[...],
                   preferred_element_type=jnp.float32))
    o_ref[...] = jnp.maximum(r, 0.0) if relu else r

  return pl.pallas_call(
      mmbody,
      grid=(grid,),
      in_specs=[
          pl.BlockSpec((2, BN, D), lambda i: (0, i, 0)),
          pl.BlockSpec((NW, BN), lambda i: (0, i)),
          pl.BlockSpec((BN, D), lambda i: (i, 0)),
          pl.BlockSpec((D, D), lambda i: (0, 0)),
          pl.BlockSpec((D, D), lambda i: (0, 0)),
          pl.BlockSpec((1, D), lambda i: (0, 0)),
      ],
      out_specs=pl.BlockSpec((BN, D), lambda i: (i, 0)),
      out_shape=jax.ShapeDtypeStruct((N, D), jnp.float32),
  )(acc, cnt2, xin, wl, wr, b.reshape(1, D))


@jax.jit
def kernel(x, edge_index, W1_l, W1_r, b1, W2_l, W2_r, b2):
  src = edge_index[0].reshape(NW, NCH, 1, CH)
  dst = edge_index[1].reshape(NW, NCH, 1, CH)
  z_hbm = jnp.zeros((ZB, D), jnp.float32)
  zc_hbm = jnp.zeros((NACC,), jnp.float32)

  acc1, cnt2 = _agg_with_cnt(x, src, dst, z_hbm, zc_hbm)
  h = _mm(acc1, cnt2, x, W1_l, W1_r, b1, relu=True)
  (acc2,) = _agg_no_cnt(h, src, dst, z_hbm, zc_hbm)
  out = _mm(acc2, cnt2, h, W2_l, W2_r, b2, relu=False)
  return out
